# Initial kernel scaffold; baseline (speedup 1.0000x reference)
import jax
import jax.numpy as jnp
from jax.experimental import pallas as pl


def kernel(x, weight):
    raise NotImplementedError("write your pallas kernel here")


# speedup vs baseline: 1.4100x; 1.4100x over previous
import functools

import jax
import jax.numpy as jnp
from jax import lax
from jax.experimental import pallas as pl
from jax.experimental.pallas import tpu as pltpu
from jax.experimental.pallas import tpu_sc as plsc

_OUT_D = 128
_L = 16384
_NC = 2
_NS = 16
_NW = _NC * _NS
_PER_W = _L // _NW
_CHUNK = 128
_NCHUNK = _PER_W // _CHUNK
_NVEC = _OUT_D // 16


def _sc_partial_sums(idx_hbm, table_hbm, out_hbm, idx_v, rows_v, acc_v, sem):
    wid = lax.axis_index("s") * _NC + lax.axis_index("c")
    pltpu.sync_copy(idx_hbm.at[wid], idx_v)
    copies = [
        pltpu.async_copy(table_hbm.at[idx_v.at[k]], rows_v.at[k], sem)
        for k in range(_NCHUNK)
    ]
    for c in copies:
        c.wait()

    def body(i, accs):
        c = i // _CHUNK
        r = lax.rem(i, _CHUNK)
        return tuple(
            accs[j] + rows_v[c, r, pl.ds(j * 16, 16)] for j in range(_NVEC)
        )

    zero = jnp.zeros((16,), jnp.float32)
    accs = lax.fori_loop(0, _PER_W, body, (zero,) * _NVEC)
    for j in range(_NVEC):
        acc_v[pl.ds(j * 16, 16)] = accs[j]
    pltpu.sync_copy(acc_v, out_hbm.at[wid])


def _tc_finish(p_ref, o_ref):
    o_ref[...] = jnp.sum(p_ref[...], axis=0, keepdims=True) * (1.0 / _L)


def kernel(x, weight):
    idx = x.astype(jnp.int32).reshape(_NW, _NCHUNK, _CHUNK)
    mesh = plsc.VectorSubcoreMesh(core_axis_name="c", subcore_axis_name="s")
    partial = pl.kernel(
        _sc_partial_sums,
        mesh=mesh,
        out_type=jax.ShapeDtypeStruct((_NW, _OUT_D), jnp.float32),
        scratch_types=[
            pltpu.VMEM((_NCHUNK, _CHUNK), jnp.int32),
            pltpu.VMEM((_NCHUNK, _CHUNK, _OUT_D), jnp.float32),
            pltpu.VMEM((_OUT_D,), jnp.float32),
            pltpu.SemaphoreType.DMA,
        ],
    )(idx, weight)
    return pl.pallas_call(
        _tc_finish,
        out_shape=jax.ShapeDtypeStruct((1, _OUT_D), jnp.float32),
    )(partial)
